# baseline (speedup 1.0000x reference)
import functools

import jax
import jax.numpy as jnp
from jax import lax
from jax.experimental import pallas as pl
from jax.experimental.pallas import tpu as pltpu
from jax.experimental.pallas import tpu_sc as plsc

N_TASKS = 80
TASK_DIM = 96
BATCH = 16384
OBS_DIM = 512
OUT_DIM = OBS_DIM + TASK_DIM


def _renorm_body(w_ref, out_ref):
    w = w_ref[...]
    ss = jnp.sum(w * w, axis=1, keepdims=True)
    scale = jnp.where(ss > 1.0, lax.rsqrt(ss), 1.0)
    out_ref[...] = w * scale


def _renorm_table(w):
    return pl.pallas_call(
        _renorm_body,
        out_shape=jax.ShapeDtypeStruct((N_TASKS, TASK_DIM), jnp.float32),
    )(w)


_info = plsc.get_sparse_core_info()
_NC = _info.num_cores
_NS = _info.num_subcores
_NW = _NC * _NS
_B_PER_W = BATCH // _NW
_CH = 128


@functools.partial(
    pl.kernel,
    mesh=plsc.VectorSubcoreMesh(core_axis_name="c", subcore_axis_name="s"),
    out_type=jax.ShapeDtypeStruct((BATCH, OUT_DIM), jnp.float32),
    compiler_params=pltpu.CompilerParams(use_tc_tiling_on_sc=False),
    scratch_types=[
        pltpu.VMEM((_B_PER_W,), jnp.int32),
        pltpu.VMEM((_B_PER_W, TASK_DIM), jnp.float32),
        pltpu.VMEM((_CH, OBS_DIM), jnp.float32),
        pltpu.SemaphoreType.DMA,
    ],
)
def _sc_assemble(obs_hbm, task_hbm, table_hbm, out_hbm, idx_v, emb_v, obs_v, sem):
    wid = lax.axis_index("s") * _NC + lax.axis_index("c")
    base = wid * _B_PER_W
    pltpu.sync_copy(task_hbm.at[pl.ds(base, _B_PER_W)], idx_v)
    pltpu.async_copy(table_hbm.at[idx_v], emb_v, sem).wait()
    pltpu.sync_copy(
        emb_v, out_hbm.at[pl.ds(base, _B_PER_W), pl.ds(OBS_DIM, TASK_DIM)]
    )
    for c in range(_B_PER_W // _CH):
        r0 = base + c * _CH
        pltpu.sync_copy(obs_hbm.at[pl.ds(r0, _CH), :], obs_v)
        pltpu.sync_copy(obs_v, out_hbm.at[pl.ds(r0, _CH), pl.ds(0, OBS_DIM)])


def kernel(obs, task, task_emb_weight):
    table_rn = _renorm_table(task_emb_weight)
    return _sc_assemble(obs, task, table_rn)
